# Initial kernel scaffold; baseline (speedup 1.0000x reference)
import functools

import jax
import jax.numpy as jnp
from jax import lax
from jax.experimental import pallas as pl
from jax.experimental.pallas import tpu as pltpu
from jax.experimental.pallas import tpu_sc as plsc

EOS = 1e-10
N = 10000
E = 160000
P = 80000
DIN = 256
EMB = 128
C = 16
TAU = 0.1

NC = 2
NS = 16
NW = NC * NS
CH = 128
PCH = P // CH
ECH = E // CH
NB = 10
BN = N // NB

f32 = jnp.float32
i32 = jnp.int32


def _prep_body(feat_ref, emb_ref, wlT_ref, bl_ref, wmT_ref, ehat_ref, g_ref):
    h = jnp.maximum(
        jnp.dot(feat_ref[...], wlT_ref[...], preferred_element_type=f32)
        + bl_ref[...], 0.0)
    g_ref[...] = jnp.dot(h, wmT_ref[...], preferred_element_type=f32)
    e = emb_ref[...]
    nrm = jnp.sqrt(jnp.sum(e * e, axis=1, keepdims=True))
    ehat_ref[...] = e / jnp.maximum(nrm, 1e-8)


def _prep(features, embedding, wlT, bl, wmT):
    return pl.pallas_call(
        _prep_body,
        grid=(NB,),
        in_specs=[
            pl.BlockSpec((BN, DIN), lambda i: (i, 0)),
            pl.BlockSpec((BN, EMB), lambda i: (i, 0)),
            pl.BlockSpec((DIN, EMB), lambda i: (0, 0)),
            pl.BlockSpec((1, EMB), lambda i: (0, 0)),
            pl.BlockSpec((EMB, C), lambda i: (0, 0)),
        ],
        out_specs=[
            pl.BlockSpec((BN, EMB), lambda i: (i, 0)),
            pl.BlockSpec((BN, C), lambda i: (i, 0)),
        ],
        out_shape=[
            jax.ShapeDtypeStruct((N, EMB), f32),
            jax.ShapeDtypeStruct((N, C), f32),
        ],
    )(features, embedding, wlT, bl, wmT)


def _sc_a_body(ehat, psrc, pdst, esrc, edst, wlp,
               wpred_out, dego_out, degi_out,
               idx0, idx1, wbuf, rows0, rows1, zbuf, dego_sh, degi_sh, sem):
    cid = lax.axis_index("c")
    sid = lax.axis_index("s")
    wid = sid * NC + cid

    @pl.when(sid == 0)
    def _():
        @pl.loop(0, N // 16)
        def _(i):
            zbuf[pl.ds(i * 16, 16)] = jnp.zeros((16,), f32)
        pltpu.sync_copy(zbuf, dego_sh)
        pltpu.sync_copy(zbuf, degi_sh)

    plsc.subcore_barrier()

    @pl.loop(wid, PCH, step=NW)
    def _(k):
        base = k * CH
        pltpu.sync_copy(psrc.at[pl.ds(base, CH)], idx0)
        pltpu.sync_copy(pdst.at[pl.ds(base, CH)], idx1)
        pltpu.async_copy(ehat.at[idx0], rows0, sem).wait()
        pltpu.async_copy(ehat.at[idx1], rows1, sem).wait()

        @pl.loop(0, CH)
        def _(e):
            acc = rows0[e, pl.ds(0, 16)] * rows1[e, pl.ds(0, 16)]
            for d in range(1, EMB // 16):
                acc = acc + (rows0[e, pl.ds(d * 16, 16)]
                             * rows1[e, pl.ds(d * 16, 16)])
            cosv = jnp.sum(acc)
            wbuf[e] = jnp.where(cosv < TAU, 0.0, cosv + EOS)

        pltpu.sync_copy(wbuf, wpred_out.at[pl.ds(base, CH)])
        pltpu.sync_copy(wbuf, dego_sh.at[idx0], add=True)
        pltpu.sync_copy(wbuf, degi_sh.at[idx1], add=True)

    @pl.loop(wid, ECH, step=NW)
    def _(k):
        base = k * CH
        pltpu.sync_copy(esrc.at[pl.ds(base, CH)], idx0)
        pltpu.sync_copy(edst.at[pl.ds(base, CH)], idx1)
        pltpu.sync_copy(wlp.at[pl.ds(base, CH)], wbuf)

        @pl.loop(0, CH // 16)
        def _(j):
            wbuf[pl.ds(j * 16, 16)] = wbuf[pl.ds(j * 16, 16)] + EOS

        pltpu.sync_copy(wbuf, dego_sh.at[idx0], add=True)
        pltpu.sync_copy(wbuf, degi_sh.at[idx1], add=True)

    plsc.subcore_barrier()

    @pl.when(sid == 0)
    def _():
        pltpu.sync_copy(dego_sh, dego_out.at[cid])
        pltpu.sync_copy(degi_sh, degi_out.at[cid])


_sc_a = functools.partial(
    pl.kernel,
    out_type=[
        jax.ShapeDtypeStruct((P,), f32),
        jax.ShapeDtypeStruct((NC, N), f32),
        jax.ShapeDtypeStruct((NC, N), f32),
    ],
    mesh=plsc.VectorSubcoreMesh(core_axis_name="c", subcore_axis_name="s",
                                num_cores=NC, num_subcores=NS),
    scratch_types=[
        pltpu.VMEM((CH,), i32),
        pltpu.VMEM((CH,), i32),
        pltpu.VMEM((CH,), f32),
        pltpu.VMEM((CH, EMB), f32),
        pltpu.VMEM((CH, EMB), f32),
        pltpu.VMEM((N,), f32),
        pltpu.VMEM_SHARED((N,), f32),
        pltpu.VMEM_SHARED((N,), f32),
        pltpu.SemaphoreType.DMA,
    ],
)(_sc_a_body)


def _deg_body(degpo_ref, degpi_ref, g_ref, gsc_ref, dinvi_ref):
    deg_o = (degpo_ref[:, 0:1] + degpo_ref[:, 1:2]) + (1.0 + EOS)
    deg_i = (degpi_ref[:, 0:1] + degpi_ref[:, 1:2]) + (1.0 + EOS)
    gsc_ref[...] = g_ref[...] * lax.rsqrt(deg_o)
    dinvi_ref[...] = lax.rsqrt(deg_i)


def _deg(degpo_t, degpi_t, g):
    return pl.pallas_call(
        _deg_body,
        out_shape=[
            jax.ShapeDtypeStruct((N, C), f32),
            jax.ShapeDtypeStruct((N, 1), f32),
        ],
    )(degpo_t, degpi_t, g)


def _sc_b_body(gsc, psrc, pdst, esrc, edst, wlp, wpred,
               acc_out, idxs, idxd, wbuf, rows, zrows, acc_sh, sem):
    cid = lax.axis_index("c")
    sid = lax.axis_index("s")
    wid = sid * NC + cid
    rows_per_tile = N // NS

    @pl.loop(0, rows_per_tile)
    def _(i):
        zrows[i, :] = jnp.zeros((16,), f32)

    pltpu.sync_copy(zrows, acc_sh.at[pl.ds(sid * rows_per_tile, rows_per_tile)])
    plsc.subcore_barrier()

    @pl.loop(wid, PCH, step=NW)
    def _(k):
        base = k * CH
        pltpu.sync_copy(psrc.at[pl.ds(base, CH)], idxs)
        pltpu.sync_copy(pdst.at[pl.ds(base, CH)], idxd)
        pltpu.sync_copy(wpred.at[pl.ds(base, CH)], wbuf)
        pltpu.async_copy(gsc.at[idxs], rows, sem).wait()

        @pl.loop(0, CH)
        def _(e):
            rows[e, :] = rows[e, :] * wbuf[e]

        pltpu.sync_copy(rows, acc_sh.at[idxd], add=True)

    @pl.loop(wid, ECH, step=NW)
    def _(k):
        base = k * CH
        pltpu.sync_copy(esrc.at[pl.ds(base, CH)], idxs)
        pltpu.sync_copy(edst.at[pl.ds(base, CH)], idxd)
        pltpu.sync_copy(wlp.at[pl.ds(base, CH)], wbuf)
        pltpu.async_copy(gsc.at[idxs], rows, sem).wait()

        @pl.loop(0, CH)
        def _(e):
            rows[e, :] = rows[e, :] * (wbuf[e] + EOS)

        pltpu.sync_copy(rows, acc_sh.at[idxd], add=True)

    plsc.subcore_barrier()
    pltpu.sync_copy(acc_sh.at[pl.ds(sid * rows_per_tile, rows_per_tile)],
                    acc_out.at[cid, pl.ds(sid * rows_per_tile, rows_per_tile)])


_sc_b = functools.partial(
    pl.kernel,
    out_type=jax.ShapeDtypeStruct((NC, N, C), f32),
    mesh=plsc.VectorSubcoreMesh(core_axis_name="c", subcore_axis_name="s",
                                num_cores=NC, num_subcores=NS),
    scratch_types=[
        pltpu.VMEM((CH,), i32),
        pltpu.VMEM((CH,), i32),
        pltpu.VMEM((CH,), f32),
        pltpu.VMEM((CH, C), f32),
        pltpu.VMEM((N // NS, C), f32),
        pltpu.VMEM_SHARED((N, C), f32),
        pltpu.SemaphoreType.DMA,
    ],
)(_sc_b_body)


def _fin_body(acc0_ref, acc1_ref, gsc_ref, dinvi_ref, bm_ref, out_ref):
    s = acc0_ref[...] + acc1_ref[...] + (1.0 + EOS) * gsc_ref[...]
    out_ref[...] = s * dinvi_ref[...] + bm_ref[...]


def _fin(acc0, acc1, gsc, dinvi, bm):
    return pl.pallas_call(
        _fin_body,
        out_shape=jax.ShapeDtypeStruct((N, C), f32),
    )(acc0, acc1, gsc, dinvi, bm)


def kernel(features, embedding, weights_lp, W_lin, b_lin, W_mlp, b_mlp,
           edges, pred_edge_index):
    edges = edges.astype(i32)
    pei = pred_edge_index.astype(i32)
    esrc, edst = edges[0], edges[1]
    psrc, pdst = pei[0], pei[1]
    wlp = weights_lp.astype(f32)

    ehat, g = _prep(features.astype(f32), embedding.astype(f32),
                    W_lin.astype(f32).T, b_lin.astype(f32).reshape(1, EMB),
                    W_mlp.astype(f32).T)

    wpred, degpo, degpi = _sc_a(ehat, psrc, pdst, esrc, edst, wlp)

    gsc, dinvi = _deg(degpo.T, degpi.T, g)

    accp = _sc_b(gsc, psrc, pdst, esrc, edst, wlp, wpred)

    return _fin(accp[0], accp[1], gsc, dinvi,
                b_mlp.astype(f32).reshape(1, C))


# speedup vs baseline: 10.6214x; 10.6214x over previous
import functools

import jax
import jax.numpy as jnp
from jax import lax
from jax.experimental import pallas as pl
from jax.experimental.pallas import tpu as pltpu
from jax.experimental.pallas import tpu_sc as plsc

EOS = 1e-10
N = 10000
E = 160000
P = 80000
DIN = 256
EMB = 128
C = 16
TAU = 0.1

NC = 2
NS = 16
NW = NC * NS
CH = 128
PCH = P // CH
ECH = E // CH
NB = 10
BN = N // NB

f32 = jnp.float32
i32 = jnp.int32


def _prep_body(feat_ref, emb_ref, wlT_ref, bl_ref, wmT_ref, ehat_ref, g_ref):
    h = jnp.maximum(
        jnp.dot(feat_ref[...], wlT_ref[...], preferred_element_type=f32)
        + bl_ref[...], 0.0)
    g_ref[...] = jnp.dot(h, wmT_ref[...], preferred_element_type=f32)
    e = emb_ref[...]
    nrm = jnp.sqrt(jnp.sum(e * e, axis=1, keepdims=True))
    ehat_ref[...] = e / jnp.maximum(nrm, 1e-8)


def _prep(features, embedding, wlT, bl, wmT):
    return pl.pallas_call(
        _prep_body,
        grid=(NB,),
        in_specs=[
            pl.BlockSpec((BN, DIN), lambda i: (i, 0)),
            pl.BlockSpec((BN, EMB), lambda i: (i, 0)),
            pl.BlockSpec((DIN, EMB), lambda i: (0, 0)),
            pl.BlockSpec((1, EMB), lambda i: (0, 0)),
            pl.BlockSpec((EMB, C), lambda i: (0, 0)),
        ],
        out_specs=[
            pl.BlockSpec((BN, EMB), lambda i: (i, 0)),
            pl.BlockSpec((BN, C), lambda i: (i, 0)),
        ],
        out_shape=[
            jax.ShapeDtypeStruct((N, EMB), f32),
            jax.ShapeDtypeStruct((N, C), f32),
        ],
    )(features, embedding, wlT, bl, wmT)


def _sc_a_body(ehat, psrc, pdst, esrc, edst, wlp, zeros1,
               wpred_out, dego_out, degi_out,
               idx0, idx1, wbuf, rows0, rows1, dego_sh, degi_sh, sem):
    cid = lax.axis_index("c")
    sid = lax.axis_index("s")
    wid = sid * NC + cid

    @pl.when(sid == 0)
    def _():
        pltpu.sync_copy(zeros1, dego_sh)
        pltpu.sync_copy(zeros1, degi_sh)

    plsc.subcore_barrier()

    @pl.loop(wid, PCH, step=NW)
    def _(k):
        base = k * CH
        pltpu.sync_copy(psrc.at[pl.ds(base, CH)], idx0)
        pltpu.sync_copy(pdst.at[pl.ds(base, CH)], idx1)
        pltpu.async_copy(ehat.at[idx0], rows0, sem).wait()
        pltpu.async_copy(ehat.at[idx1], rows1, sem).wait()

        lanes = lax.iota(i32, 16)

        @pl.loop(0, CH // 16)
        def _(j):
            wv = jnp.zeros((16,), f32)
            for k in range(16):
                e = j * 16 + k
                acc = rows0[e, pl.ds(0, 16)] * rows1[e, pl.ds(0, 16)]
                for d in range(1, EMB // 16):
                    acc = acc + (rows0[e, pl.ds(d * 16, 16)]
                                 * rows1[e, pl.ds(d * 16, 16)])
                cosv = jnp.sum(acc)
                we = jnp.where(cosv < TAU, 0.0, cosv + EOS)
                wv = jnp.where(lanes == k, we, wv)
            wbuf[pl.ds(j * 16, 16)] = wv

        pltpu.sync_copy(wbuf, wpred_out.at[pl.ds(base, CH)])
        pltpu.sync_copy(wbuf, dego_sh.at[idx0], add=True)
        pltpu.sync_copy(wbuf, degi_sh.at[idx1], add=True)

    @pl.loop(wid, ECH, step=NW)
    def _(k):
        base = k * CH
        pltpu.sync_copy(esrc.at[pl.ds(base, CH)], idx0)
        pltpu.sync_copy(edst.at[pl.ds(base, CH)], idx1)
        pltpu.sync_copy(wlp.at[pl.ds(base, CH)], wbuf)

        @pl.loop(0, CH // 16)
        def _(j):
            wbuf[pl.ds(j * 16, 16)] = wbuf[pl.ds(j * 16, 16)] + EOS

        pltpu.sync_copy(wbuf, dego_sh.at[idx0], add=True)
        pltpu.sync_copy(wbuf, degi_sh.at[idx1], add=True)

    plsc.subcore_barrier()

    @pl.when(sid == 0)
    def _():
        pltpu.sync_copy(dego_sh, dego_out.at[cid])
        pltpu.sync_copy(degi_sh, degi_out.at[cid])


_sc_a = functools.partial(
    pl.kernel,
    out_type=[
        jax.ShapeDtypeStruct((P,), f32),
        jax.ShapeDtypeStruct((NC, N), f32),
        jax.ShapeDtypeStruct((NC, N), f32),
    ],
    mesh=plsc.VectorSubcoreMesh(core_axis_name="c", subcore_axis_name="s",
                                num_cores=NC, num_subcores=NS),
    scratch_types=[
        pltpu.VMEM((CH,), i32),
        pltpu.VMEM((CH,), i32),
        pltpu.VMEM((CH,), f32),
        pltpu.VMEM((CH, EMB), f32),
        pltpu.VMEM((CH, EMB), f32),
        pltpu.VMEM_SHARED((N,), f32),
        pltpu.VMEM_SHARED((N,), f32),
        pltpu.SemaphoreType.DMA,
    ],
    compiler_params=pltpu.CompilerParams(needs_layout_passes=False),
)(_sc_a_body)


def _deg_body(degpo_ref, degpi_ref, g_ref, gsc_ref, dinvi_ref):
    deg_o = (degpo_ref[:, 0:1] + degpo_ref[:, 1:2]) + (1.0 + EOS)
    deg_i = (degpi_ref[:, 0:1] + degpi_ref[:, 1:2]) + (1.0 + EOS)
    gsc_ref[:, 0:C] = g_ref[...] * lax.rsqrt(deg_o)
    gsc_ref[:, C:EMB] = jnp.zeros((N, EMB - C), f32)
    dinvi_ref[...] = lax.rsqrt(deg_i)


def _deg(degpo_t, degpi_t, g):
    return pl.pallas_call(
        _deg_body,
        out_shape=[
            jax.ShapeDtypeStruct((N, EMB), f32),
            jax.ShapeDtypeStruct((N, 1), f32),
        ],
    )(degpo_t, degpi_t, g)


def _sc_b_body(gsc, psrc, pdst, esrc, edst, wlp, wpred, zeros2,
               acc_out, idxs, idxd, wbuf, rows, acc_sh, sem):
    cid = lax.axis_index("c")
    sid = lax.axis_index("s")
    wid = sid * NC + cid

    @pl.when(sid == 0)
    def _():
        pltpu.sync_copy(zeros2, acc_sh)

    plsc.subcore_barrier()

    @pl.loop(wid, PCH, step=NW)
    def _(k):
        base = k * CH
        pltpu.sync_copy(psrc.at[pl.ds(base, CH)], idxs)
        pltpu.sync_copy(pdst.at[pl.ds(base, CH)], idxd)
        pltpu.sync_copy(wpred.at[pl.ds(base, CH)], wbuf)
        pltpu.async_copy(gsc.at[idxs], rows, sem).wait()

        for j in range(CH // 16):
            wv = wbuf[pl.ds(j * 16, 16)]
            for k in range(16):
                e = j * 16 + k
                rows[e, pl.ds(0, 16)] = rows[e, pl.ds(0, 16)] * wv[k]

        pltpu.sync_copy(rows, acc_sh.at[idxd], add=True)

    @pl.loop(wid, ECH, step=NW)
    def _(k):
        base = k * CH
        pltpu.sync_copy(esrc.at[pl.ds(base, CH)], idxs)
        pltpu.sync_copy(edst.at[pl.ds(base, CH)], idxd)
        pltpu.sync_copy(wlp.at[pl.ds(base, CH)], wbuf)
        pltpu.async_copy(gsc.at[idxs], rows, sem).wait()

        for j in range(CH // 16):
            wv = wbuf[pl.ds(j * 16, 16)] + EOS
            for k in range(16):
                e = j * 16 + k
                rows[e, pl.ds(0, 16)] = rows[e, pl.ds(0, 16)] * wv[k]

        pltpu.sync_copy(rows, acc_sh.at[idxd], add=True)

    plsc.subcore_barrier()

    @pl.when(sid == 0)
    def _():
        pltpu.sync_copy(acc_sh, acc_out.at[cid])


_sc_b = functools.partial(
    pl.kernel,
    out_type=jax.ShapeDtypeStruct((NC, N, EMB), f32),
    mesh=plsc.VectorSubcoreMesh(core_axis_name="c", subcore_axis_name="s",
                                num_cores=NC, num_subcores=NS),
    scratch_types=[
        pltpu.VMEM((CH,), i32),
        pltpu.VMEM((CH,), i32),
        pltpu.VMEM((CH,), f32),
        pltpu.VMEM((CH, EMB), f32),
        pltpu.VMEM_SHARED((N, EMB), f32),
        pltpu.SemaphoreType.DMA,
    ],
    compiler_params=pltpu.CompilerParams(needs_layout_passes=False),
)(_sc_b_body)


def _fin_body(acc0_ref, acc1_ref, gsc_ref, dinvi_ref, bm_ref, out_ref):
    s = (acc0_ref[:, 0:C] + acc1_ref[:, 0:C]
         + (1.0 + EOS) * gsc_ref[:, 0:C])
    out_ref[...] = s * dinvi_ref[...] + bm_ref[...]


def _fin(acc0, acc1, gsc, dinvi, bm):
    return pl.pallas_call(
        _fin_body,
        out_shape=jax.ShapeDtypeStruct((N, C), f32),
    )(acc0, acc1, gsc, dinvi, bm)


def kernel(features, embedding, weights_lp, W_lin, b_lin, W_mlp, b_mlp,
           edges, pred_edge_index):
    edges = edges.astype(i32)
    pei = pred_edge_index.astype(i32)
    esrc, edst = edges[0], edges[1]
    psrc, pdst = pei[0], pei[1]
    wlp = weights_lp.astype(f32)

    ehat, g = _prep(features.astype(f32), embedding.astype(f32),
                    W_lin.astype(f32).T, b_lin.astype(f32).reshape(1, EMB),
                    W_mlp.astype(f32).T)

    wpred, degpo, degpi = _sc_a(ehat, psrc, pdst, esrc, edst, wlp,
                                jnp.zeros((N,), f32))

    gsc, dinvi = _deg(degpo.T, degpi.T, g)

    accp = _sc_b(gsc, psrc, pdst, esrc, edst, wlp, wpred,
                 jnp.zeros((N, EMB), f32))

    return _fin(accp[0], accp[1], gsc, dinvi,
                b_mlp.astype(f32).reshape(1, C))


# speedup vs baseline: 12.0620x; 1.1356x over previous
import functools

import jax
import jax.numpy as jnp
from jax import lax
from jax.experimental import pallas as pl
from jax.experimental.pallas import tpu as pltpu
from jax.experimental.pallas import tpu_sc as plsc

EOS = 1e-10
N = 10000
E = 160000
P = 80000
DIN = 256
EMB = 128
C = 16
TAU = 0.1

NC = 2
NS = 16
NW = NC * NS
CH = 128
PCH = P // CH
ECH = E // CH
TCH = (P + E) // CH
NB = 10
BN = N // NB

f32 = jnp.float32
i32 = jnp.int32


def _prep_body(feat_ref, emb_ref, wlT_ref, bl_ref, wmT_ref, wlp_ref,
               ehat_ref, g_ref, wlpe_ref):
    h = jnp.maximum(
        jnp.dot(feat_ref[...], wlT_ref[...], preferred_element_type=f32)
        + bl_ref[...], 0.0)
    g_ref[...] = jnp.dot(h, wmT_ref[...], preferred_element_type=f32)
    e = emb_ref[...]
    nrm = jnp.sqrt(jnp.sum(e * e, axis=1, keepdims=True))
    ehat_ref[...] = e / jnp.maximum(nrm, 1e-8)
    wlpe_ref[...] = wlp_ref[...] + EOS


def _prep(features, embedding, wlT, bl, wmT, wlp2):
    return pl.pallas_call(
        _prep_body,
        grid=(NB,),
        in_specs=[
            pl.BlockSpec((BN, DIN), lambda i: (i, 0)),
            pl.BlockSpec((BN, EMB), lambda i: (i, 0)),
            pl.BlockSpec((DIN, EMB), lambda i: (0, 0)),
            pl.BlockSpec((1, EMB), lambda i: (0, 0)),
            pl.BlockSpec((EMB, C), lambda i: (0, 0)),
            pl.BlockSpec((1, 8, E // NB // 8), lambda i: (i, 0, 0)),
        ],
        out_specs=[
            pl.BlockSpec((BN, EMB), lambda i: (i, 0)),
            pl.BlockSpec((BN, C), lambda i: (i, 0)),
            pl.BlockSpec((1, 8, E // NB // 8), lambda i: (i, 0, 0)),
        ],
        out_shape=[
            jax.ShapeDtypeStruct((N, EMB), f32),
            jax.ShapeDtypeStruct((N, C), f32),
            jax.ShapeDtypeStruct((NB, 8, E // NB // 8), f32),
        ],
    )(features, embedding, wlT, bl, wmT, wlp2)


def _cos_chunk(rows0, rows1, wbuf):
    lanes = lax.iota(i32, 16)

    @pl.loop(0, CH // 16)
    def _(j):
        wv = jnp.zeros((16,), f32)
        for q in range(16):
            e = j * 16 + q
            acc = rows0[e, pl.ds(0, 16)] * rows1[e, pl.ds(0, 16)]
            for d in range(1, EMB // 16):
                acc = acc + (rows0[e, pl.ds(d * 16, 16)]
                             * rows1[e, pl.ds(d * 16, 16)])
            cosv = jnp.sum(acc)
            we = jnp.where(cosv < TAU, 0.0, cosv + EOS)
            wv = jnp.where(lanes == q, we, wv)
        wbuf[pl.ds(j * 16, 16)] = wv


def _sc_a_body(ehat, psrc, pdst, esrc, edst, wlpe, zeros1,
               wpred_out, dego_out, degi_out,
               idx0a, idx1a, wba, rows0a, rows1a,
               idx0b, idx1b, wbb, rows0b, rows1b,
               dego_sh, degi_sh,
               gsa, gsb, wsa, wsb, dsa0, dsa1, dsb0, dsb1):
    cid = lax.axis_index("c")
    sid = lax.axis_index("s")
    wid = sid * NC + cid

    @pl.when(sid == 0)
    def _():
        pltpu.sync_copy(zeros1, dego_sh)
        pltpu.sync_copy(zeros1, degi_sh)

    plsc.subcore_barrier()

    bufs = ((idx0a, idx1a, wba, rows0a, rows1a, gsa, wsa, dsa0, dsa1),
            (idx0b, idx1b, wbb, rows0b, rows1b, gsb, wsb, dsb0, dsb1))

    def load_fire_pred(k, i0, i1, r0, r1, gs):
        base = k * CH
        pltpu.sync_copy(psrc.at[pl.ds(base, CH)], i0)
        pltpu.sync_copy(pdst.at[pl.ds(base, CH)], i1)
        pltpu.async_copy(ehat.at[i0], r0, gs)
        pltpu.async_copy(ehat.at[i1], r1, gs)

    nt = (PCH - wid + NW - 1) // NW
    load_fire_pred(wid, idx0a, idx1a, rows0a, rows1a, gsa)

    @pl.loop(0, nt)
    def _(t):
        for par in range(2):
            @pl.when(t % 2 == par)
            def _():
                i0, i1, wb, r0, r1, gs, ws, d0, d1 = bufs[par]
                i0n, i1n, wbn, r0n, r1n, gsn, wsn, d0n, d1n = bufs[1 - par]
                k = wid + t * NW
                base = k * CH
                pltpu.make_async_copy(ehat.at[i0], r0, gs).wait()
                pltpu.make_async_copy(ehat.at[i1], r1, gs).wait()
                _cos_chunk(r0, r1, wb)

                @pl.when(t + 1 < nt)
                def _():
                    @pl.when(t >= 1)
                    def _():
                        pltpu.make_async_copy(
                            wbn, wpred_out.at[pl.ds(0, CH)], wsn).wait()
                        pltpu.make_async_copy(
                            wbn, dego_sh.at[i0n], d0n).wait()
                        pltpu.make_async_copy(
                            wbn, degi_sh.at[i1n], d1n).wait()
                    load_fire_pred(k + NW, i0n, i1n, r0n, r1n, gsn)

                pltpu.async_copy(wb, wpred_out.at[pl.ds(base, CH)], ws)
                pltpu.async_copy(wb, dego_sh.at[i0], d0, add=True)
                pltpu.async_copy(wb, degi_sh.at[i1], d1, add=True)

    for par in range(2):
        i0, i1, wb, r0, r1, gs, ws, d0, d1 = bufs[par]
        pltpu.make_async_copy(wb, wpred_out.at[pl.ds(0, CH)], ws).wait()
        pltpu.make_async_copy(wb, dego_sh.at[i0], d0).wait()
        pltpu.make_async_copy(wb, degi_sh.at[i1], d1).wait()

    ne = (ECH - wid + NW - 1) // NW

    def load_edge(k, i0, i1, wb):
        base = k * CH
        pltpu.sync_copy(esrc.at[pl.ds(base, CH)], i0)
        pltpu.sync_copy(edst.at[pl.ds(base, CH)], i1)
        pltpu.sync_copy(wlpe.at[pl.ds(base, CH)], wb)

    load_edge(wid, idx0a, idx1a, wba)

    @pl.loop(0, ne)
    def _(t):
        for par in range(2):
            @pl.when(t % 2 == par)
            def _():
                i0, i1, wb, r0, r1, gs, ws, d0, d1 = bufs[par]
                i0n, i1n, wbn, r0n, r1n, gsn, wsn, d0n, d1n = bufs[1 - par]
                k = wid + t * NW
                pltpu.async_copy(wb, dego_sh.at[i0], d0, add=True)
                pltpu.async_copy(wb, degi_sh.at[i1], d1, add=True)

                @pl.when(t + 1 < ne)
                def _():
                    @pl.when(t >= 1)
                    def _():
                        pltpu.make_async_copy(
                            wbn, dego_sh.at[i0n], d0n).wait()
                        pltpu.make_async_copy(
                            wbn, degi_sh.at[i1n], d1n).wait()
                    load_edge(k + NW, i0n, i1n, wbn)

    for par in range(2):
        i0, i1, wb, r0, r1, gs, ws, d0, d1 = bufs[par]
        pltpu.make_async_copy(wb, dego_sh.at[i0], d0).wait()
        pltpu.make_async_copy(wb, degi_sh.at[i1], d1).wait()

    plsc.subcore_barrier()

    @pl.when(sid == 0)
    def _():
        pltpu.sync_copy(dego_sh, dego_out.at[cid])
        pltpu.sync_copy(degi_sh, degi_out.at[cid])


_sc_a = functools.partial(
    pl.kernel,
    out_type=[
        jax.ShapeDtypeStruct((P,), f32),
        jax.ShapeDtypeStruct((NC, N), f32),
        jax.ShapeDtypeStruct((NC, N), f32),
    ],
    mesh=plsc.VectorSubcoreMesh(core_axis_name="c", subcore_axis_name="s",
                                num_cores=NC, num_subcores=NS),
    scratch_types=[
        pltpu.VMEM((CH,), i32),
        pltpu.VMEM((CH,), i32),
        pltpu.VMEM((CH,), f32),
        pltpu.VMEM((CH, EMB), f32),
        pltpu.VMEM((CH, EMB), f32),
        pltpu.VMEM((CH,), i32),
        pltpu.VMEM((CH,), i32),
        pltpu.VMEM((CH,), f32),
        pltpu.VMEM((CH, EMB), f32),
        pltpu.VMEM((CH, EMB), f32),
        pltpu.VMEM_SHARED((N,), f32),
        pltpu.VMEM_SHARED((N,), f32),
        pltpu.SemaphoreType.DMA,
        pltpu.SemaphoreType.DMA,
        pltpu.SemaphoreType.DMA,
        pltpu.SemaphoreType.DMA,
        pltpu.SemaphoreType.DMA,
        pltpu.SemaphoreType.DMA,
        pltpu.SemaphoreType.DMA,
        pltpu.SemaphoreType.DMA,
    ],
    compiler_params=pltpu.CompilerParams(needs_layout_passes=False),
)(_sc_a_body)


def _deg_body(degpo_ref, degpi_ref, g_ref, gsc_ref, dinvi_ref):
    deg_o = (degpo_ref[:, 0:1] + degpo_ref[:, 1:2]) + (1.0 + EOS)
    deg_i = (degpi_ref[:, 0:1] + degpi_ref[:, 1:2]) + (1.0 + EOS)
    gsc_ref[:, 0:C] = g_ref[...] * lax.rsqrt(deg_o)
    gsc_ref[:, C:EMB] = jnp.zeros((N, EMB - C), f32)
    dinvi_ref[...] = lax.rsqrt(deg_i)


def _deg(degpo_t, degpi_t, g):
    return pl.pallas_call(
        _deg_body,
        out_shape=[
            jax.ShapeDtypeStruct((N, EMB), f32),
            jax.ShapeDtypeStruct((N, 1), f32),
        ],
    )(degpo_t, degpi_t, g)


def _sc_b_body(gsc, srcall, dstall, wall, zeros2,
               acc_out,
               idxsa, idxda, wba, rowsa,
               idxsb, idxdb, wbb, rowsb,
               acc_sh, gsa, gsb, ssa, ssb):
    cid = lax.axis_index("c")
    sid = lax.axis_index("s")
    wid = sid * NC + cid

    @pl.when(sid == 0)
    def _():
        pltpu.sync_copy(zeros2, acc_sh)

    plsc.subcore_barrier()

    bufs = ((idxsa, idxda, wba, rowsa, gsa, ssa),
            (idxsb, idxdb, wbb, rowsb, gsb, ssb))

    def load_fire(k, isb, idb, wb, rb, gs):
        base = k * CH
        pltpu.sync_copy(srcall.at[pl.ds(base, CH)], isb)
        pltpu.sync_copy(dstall.at[pl.ds(base, CH)], idb)
        pltpu.sync_copy(wall.at[pl.ds(base, CH)], wb)
        pltpu.async_copy(gsc.at[isb], rb, gs)

    nt = (TCH - wid + NW - 1) // NW
    load_fire(wid, idxsa, idxda, wba, rowsa, gsa)

    @pl.loop(0, nt)
    def _(t):
        for par in range(2):
            @pl.when(t % 2 == par)
            def _():
                isb, idb, wb, rb, gs, ss = bufs[par]
                isn, idn, wbn, rbn, gsn, ssn = bufs[1 - par]
                k = wid + t * NW
                pltpu.make_async_copy(gsc.at[isb], rb, gs).wait()

                for j in range(CH // 16):
                    wv = wb[pl.ds(j * 16, 16)]
                    for q in range(16):
                        e = j * 16 + q
                        rb[e, pl.ds(0, 16)] = rb[e, pl.ds(0, 16)] * wv[q]

                @pl.when(t + 1 < nt)
                def _():
                    @pl.when(t >= 1)
                    def _():
                        pltpu.make_async_copy(
                            rbn, acc_sh.at[idn], ssn).wait()
                    load_fire(k + NW, isn, idn, wbn, rbn, gsn)

                pltpu.async_copy(rb, acc_sh.at[idb], ss, add=True)

    for par in range(2):
        isb, idb, wb, rb, gs, ss = bufs[par]
        pltpu.make_async_copy(rb, acc_sh.at[idb], ss).wait()

    plsc.subcore_barrier()

    @pl.when(sid == 0)
    def _():
        pltpu.sync_copy(acc_sh, acc_out.at[cid])


_sc_b = functools.partial(
    pl.kernel,
    out_type=jax.ShapeDtypeStruct((NC, N, EMB), f32),
    mesh=plsc.VectorSubcoreMesh(core_axis_name="c", subcore_axis_name="s",
                                num_cores=NC, num_subcores=NS),
    scratch_types=[
        pltpu.VMEM((CH,), i32),
        pltpu.VMEM((CH,), i32),
        pltpu.VMEM((CH,), f32),
        pltpu.VMEM((CH, EMB), f32),
        pltpu.VMEM((CH,), i32),
        pltpu.VMEM((CH,), i32),
        pltpu.VMEM((CH,), f32),
        pltpu.VMEM((CH, EMB), f32),
        pltpu.VMEM_SHARED((N, EMB), f32),
        pltpu.SemaphoreType.DMA,
        pltpu.SemaphoreType.DMA,
        pltpu.SemaphoreType.DMA,
        pltpu.SemaphoreType.DMA,
    ],
    compiler_params=pltpu.CompilerParams(needs_layout_passes=False),
)(_sc_b_body)


def _fin_body(acc0_ref, acc1_ref, gsc_ref, dinvi_ref, bm_ref, out_ref):
    s = (acc0_ref[:, 0:C] + acc1_ref[:, 0:C]
         + (1.0 + EOS) * gsc_ref[:, 0:C])
    out_ref[...] = s * dinvi_ref[...] + bm_ref[...]


def _fin(acc0, acc1, gsc, dinvi, bm):
    return pl.pallas_call(
        _fin_body,
        out_shape=jax.ShapeDtypeStruct((N, C), f32),
    )(acc0, acc1, gsc, dinvi, bm)


def kernel(features, embedding, weights_lp, W_lin, b_lin, W_mlp, b_mlp,
           edges, pred_edge_index):
    edges = edges.astype(i32)
    pei = pred_edge_index.astype(i32)
    esrc, edst = edges[0], edges[1]
    psrc, pdst = pei[0], pei[1]
    wlp = weights_lp.astype(f32)

    ehat, g, wlpe = _prep(features.astype(f32), embedding.astype(f32),
                          W_lin.astype(f32).T, b_lin.astype(f32).reshape(1, EMB),
                          W_mlp.astype(f32).T,
                          wlp.reshape(NB, 8, E // NB // 8))
    wlpe = wlpe.reshape(E)

    wpred, degpo, degpi = _sc_a(ehat, psrc, pdst, esrc, edst, wlpe,
                                jnp.zeros((N,), f32))

    gsc, dinvi = _deg(degpo.T, degpi.T, g)

    srcall = jnp.concatenate([psrc, esrc])
    dstall = jnp.concatenate([pdst, edst])
    wall = jnp.concatenate([wpred, wlpe])
    accp = _sc_b(gsc, srcall, dstall, wall, jnp.zeros((N, EMB), f32))

    return _fin(accp[0], accp[1], gsc, dinvi,
                b_mlp.astype(f32).reshape(1, C))
